# baseline (speedup 1.0000x reference)
import functools
import math

import jax
import jax.numpy as jnp
from jax import lax
from jax.experimental import pallas as pl
from jax.experimental.pallas import tpu as pltpu
from jax.experimental.pallas import tpu_sc as plsc

NC = 2
NS = 16
NW = NC * NS
L = 16

V = 1000000
D = 64
NB = 4096
NT = 200
W = 128
NBUF = 4
NOB = 2
SCALE = math.sqrt(D)

_mesh = plsc.VectorSubcoreMesh(core_axis_name="c", subcore_axis_name="s")


@functools.partial(
    pl.kernel,
    out_type=jax.ShapeDtypeStruct((NT, D, NB), jnp.float32),
    mesh=_mesh,
    scratch_types=[
        pltpu.VMEM((NT, W), jnp.int32),
        pltpu.VMEM((NBUF, W, D), jnp.float32),
        pltpu.VMEM((NOB, D, W), jnp.float32),
        pltpu.SemaphoreType.DMA((NBUF,)),
        pltpu.SemaphoreType.DMA((NOB,)),
    ],
    compiler_params=pltpu.CompilerParams(
        use_tc_tiling_on_sc=False, needs_layout_passes=False
    ),
)
def _gather_t(lut_hbm, xt_hbm, out_hbm, idx_v, rows, obuf, gsem, osem):
    wid = lax.axis_index("s") * NC + lax.axis_index("c")
    b0 = wid * W
    pltpu.sync_copy(xt_hbm.at[:, pl.ds(b0, W)], idx_v)

    dgi = [jnp.arange(16, dtype=jnp.int32) + (16 * g) for g in range(D // L)]

    for b in range(NBUF):
        pltpu.make_async_copy(
            lut_hbm.at[idx_v.at[b]], rows.at[b], gsem.at[b]
        ).start()

    @pl.loop(0, NT, step=NBUF)
    def _(t0):
        for bb in range(NBUF):
            t = t0 + bb
            q = bb % NOB
            pltpu.make_async_copy(
                lut_hbm.at[idx_v.at[t]], rows.at[bb], gsem.at[bb]
            ).wait()

            def _drain():
                pltpu.make_async_copy(
                    obuf.at[q],
                    out_hbm.at[t - NOB, :, pl.ds(b0, W)],
                    osem.at[q],
                ).wait()

            if bb >= NOB:
                _drain()
            else:
                pl.when(t0 > 0)(_drain)

            @pl.loop(0, W)
            def _(r):
                rsp = jnp.full((L,), r, dtype=jnp.int32)
                for g in range(D // L):
                    vals = rows.at[bb, r, pl.ds(g * L, L)][...] * SCALE
                    plsc.store_scatter(obuf.at[q], [dgi[g], rsp], vals)

            pltpu.make_async_copy(
                obuf.at[q], out_hbm.at[t, :, pl.ds(b0, W)], osem.at[q]
            ).start()

            @pl.when(t + NBUF < NT)
            def _():
                pltpu.make_async_copy(
                    lut_hbm.at[idx_v.at[t + NBUF]], rows.at[bb], gsem.at[bb]
                ).start()

    for b in range(NOB):
        pltpu.make_async_copy(
            obuf.at[(NT - NOB + b) % NOB],
            out_hbm.at[NT - NOB + b, :, pl.ds(b0, W)],
            osem.at[(NT - NOB + b) % NOB],
        ).wait()


def kernel(x, lut):
    xt = jnp.transpose(x).astype(jnp.int32)
    out_t = _gather_t(lut, xt)
    return jnp.transpose(out_t, (2, 0, 1))


# speedup vs baseline: 1.5198x; 1.5198x over previous
import functools
import math

import jax
import jax.numpy as jnp
from jax import lax
from jax.experimental import pallas as pl
from jax.experimental.pallas import tpu as pltpu
from jax.experimental.pallas import tpu_sc as plsc

NC = 2
NS = 16
NW = NC * NS
L = 16

V = 1000000
VP = V // 2
D = 64
DP = 128
NB = 4096
NT = 200
W = 128
NBUF = 4
NOB = 2
SCALE = math.sqrt(D)

_mesh = plsc.VectorSubcoreMesh(core_axis_name="c", subcore_axis_name="s")


@functools.partial(
    pl.kernel,
    out_type=jax.ShapeDtypeStruct((NT, D, NB), jnp.float32),
    mesh=_mesh,
    scratch_types=[
        pltpu.VMEM((NT, W), jnp.int32),
        pltpu.VMEM((NBUF, W), jnp.int32),
        pltpu.VMEM((NBUF, W, DP), jnp.float32),
        pltpu.VMEM((NOB, D, W), jnp.float32),
        pltpu.SemaphoreType.DMA((NBUF,)),
        pltpu.SemaphoreType.DMA((NOB,)),
    ],
    compiler_params=pltpu.CompilerParams(
        use_tc_tiling_on_sc=True, needs_layout_passes=False
    ),
)
def _gather_t(lut2_hbm, xt_hbm, out_hbm, idx_v, idx2_v, rows, obuf, gsem, osem):
    wid = lax.axis_index("s") * NC + lax.axis_index("c")
    b0 = wid * W
    pltpu.sync_copy(xt_hbm.at[:, pl.ds(b0, W)], idx_v)

    riota = jnp.arange(L, dtype=jnp.int32)

    def fire(t, bb):
        for g in range(W // L):
            idx2_v.at[bb, pl.ds(g * L, L)][...] = jax.lax.shift_right_logical(
                idx_v.at[t, pl.ds(g * L, L)][...], 1
            )
        pltpu.make_async_copy(
            lut2_hbm.at[idx2_v.at[bb]], rows.at[bb], gsem.at[bb]
        ).start()

    for b in range(NBUF):
        fire(b, b)

    @pl.loop(0, NT, step=NBUF)
    def _(t0):
        for bb in range(NBUF):
            t = t0 + bb
            q = bb % NOB
            pltpu.make_async_copy(
                lut2_hbm.at[idx2_v.at[bb]], rows.at[bb], gsem.at[bb]
            ).wait()

            def _drain():
                pltpu.make_async_copy(
                    obuf.at[q],
                    out_hbm.at[t - NOB, :, pl.ds(b0, W)],
                    osem.at[q],
                ).wait()

            if bb >= NOB:
                _drain()
            else:
                pl.when(t0 > 0)(_drain)

            for g in range(W // L):
                vbits = idx_v.at[t, pl.ds(g * L, L)][...]
                cb16 = jax.lax.shift_left((vbits & 1), 6)
                rg = riota + (g * L)

                @plsc.parallel_loop(0, D, unroll=8)
                def _(d):
                    vals = plsc.load_gather(rows.at[bb], [rg, cb16 + d])
                    obuf.at[q, d, pl.ds(g * L, L)][...] = vals * SCALE

            pltpu.make_async_copy(
                obuf.at[q], out_hbm.at[t, :, pl.ds(b0, W)], osem.at[q]
            ).start()

            @pl.when(t + NBUF < NT)
            def _():
                fire(t + NBUF, bb)

    for b in range(NOB):
        pltpu.make_async_copy(
            obuf.at[(NT - NOB + b) % NOB],
            out_hbm.at[NT - NOB + b, :, pl.ds(b0, W)],
            osem.at[(NT - NOB + b) % NOB],
        ).wait()


def kernel(x, lut):
    xt = jnp.transpose(x).astype(jnp.int32)
    lut2 = jnp.reshape(lut, (VP, DP))
    out_t = _gather_t(lut2, xt)
    return jnp.transpose(out_t, (2, 0, 1))
